# Initial kernel scaffold; baseline (speedup 1.0000x reference)
import jax
import jax.numpy as jnp
from jax.experimental import pallas as pl


def kernel(x, edge_index, emb, W1, b1, W2, b2):
    raise NotImplementedError("write your pallas kernel here")


# speedup vs baseline: 9.5749x; 9.5749x over previous
import functools

import jax
import jax.numpy as jnp
from jax import lax
from jax.experimental import pallas as pl
from jax.experimental.pallas import tpu as pltpu
from jax.experimental.pallas import tpu_sc as plsc

N = 10000
D = 128
H = 64
E = 320000
NB = 79
NP = NB * 128
EB = 128
NBATCH = 157
EPT = NBATCH * EB
E_PAD = EPT * 16
VPT = E_PAD // 32
RPT = NP // 16

_mesh = plsc.VectorSubcoreMesh(core_axis_name="c", subcore_axis_name="s")
_sc_params = pltpu.CompilerParams(
    needs_layout_passes=False, use_tc_tiling_on_sc=False)


@functools.partial(
    pl.kernel,
    mesh=_mesh,
    compiler_params=_sc_params,
    out_type=[
        jax.ShapeDtypeStruct((32, NP), jnp.int32),
        jax.ShapeDtypeStruct((32, NP), jnp.int32),
    ],
    scratch_types=[
        pltpu.VMEM((VPT,), jnp.int32),
        pltpu.VMEM((VPT,), jnp.int32),
        pltpu.VMEM((NP,), jnp.int32),
        pltpu.VMEM((NP,), jnp.int32),
    ],
)
def _sc_hist(src_hbm, dst_hbm, outs_hbm, outd_hbm, sbuf, dbuf, hs, hd):
    wid = lax.axis_index("s") * 2 + lax.axis_index("c")
    base = wid * VPT
    pltpu.sync_copy(src_hbm.at[pl.ds(base, VPT)], sbuf)
    pltpu.sync_copy(dst_hbm.at[pl.ds(base, VPT)], dbuf)

    zeros16 = jnp.zeros((16,), jnp.int32)

    def zero_body(i, _):
        hs[pl.ds(i * 16, 16)] = zeros16
        hd[pl.ds(i * 16, 16)] = zeros16
        return 0

    lax.fori_loop(0, NP // 16, zero_body, 0)

    ones16 = jnp.ones((16,), jnp.int32)

    def hist_body(i, _):
        s16 = sbuf[pl.ds(i * 16, 16)]
        plsc.addupdate_scatter(hs, [s16], ones16)
        d16 = dbuf[pl.ds(i * 16, 16)]
        plsc.addupdate_scatter(hd, [d16], ones16)
        return 0

    lax.fori_loop(0, VPT // 16, hist_body, 0)

    pltpu.sync_copy(hs, outs_hbm.at[wid])
    pltpu.sync_copy(hd, outd_hbm.at[wid])


def _tc_stats_body(cs_ref, cd_ref, sidx_ref, dinv_ref):
    cs = jnp.sum(cs_ref[...], axis=0)
    cd = jnp.sum(cd_ref[...], axis=0)
    row = lax.broadcasted_iota(jnp.int32, (NB, 128), 0)
    col = lax.broadcasted_iota(jnp.int32, (NB, 128), 1)
    flat = row * 128 + col
    valid = flat < N
    present = jnp.where(valid & ((cs + cd) > 0), 1.0, 0.0).astype(jnp.float32)
    ones_col = jnp.ones((128, 1), jnp.float32)
    rowsum = jnp.dot(present, ones_col, preferred_element_type=jnp.float32)
    i79 = lax.broadcasted_iota(jnp.int32, (NB, NB), 0)
    j79 = lax.broadcasted_iota(jnp.int32, (NB, NB), 1)
    mrow = jnp.where(i79 > j79, 1.0, 0.0).astype(jnp.float32)
    rowpref = jnp.dot(mrow, rowsum, preferred_element_type=jnp.float32)
    i128 = lax.broadcasted_iota(jnp.int32, (128, 128), 0)
    j128 = lax.broadcasted_iota(jnp.int32, (128, 128), 1)
    ucol = jnp.where(i128 < j128, 1.0, 0.0).astype(jnp.float32)
    wr = jnp.dot(present, ucol, preferred_element_type=jnp.float32)
    pos = (rowpref + wr).astype(jnp.int32)
    sidx_ref[...] = jnp.where(present > 0, pos, N)
    deg = (cd + 1).astype(jnp.float32)
    dinv_ref[...] = jnp.where(valid, lax.rsqrt(deg), 1.0)


def _tc_stats(cs_part, cd_part):
    return pl.pallas_call(
        _tc_stats_body,
        out_shape=[
            jax.ShapeDtypeStruct((NB, 128), jnp.int32),
            jax.ShapeDtypeStruct((NB, 128), jnp.float32),
        ],
    )(cs_part, cd_part)


@functools.partial(
    pl.kernel,
    mesh=_mesh,
    compiler_params=_sc_params,
    out_type=jax.ShapeDtypeStruct((2, NP, H), jnp.float32),
    scratch_types=[
        pltpu.VMEM((EB,), jnp.int32),
        pltpu.VMEM((EB, H), jnp.float32),
        pltpu.VMEM((EB, H), jnp.float32),
        pltpu.VMEM((1, H), jnp.float32),
        pltpu.VMEM_SHARED((NP, H), jnp.float32),
    ],
)
def _sc_unique_gather(emb_hbm, sidx_hbm, h0_hbm, idxb, rowb, initb, row0, h0_sp):
    c = lax.axis_index("c")
    s = lax.axis_index("s")
    pltpu.sync_copy(emb_hbm.at[c, pl.ds(0, 1)], row0)

    def fill_body(j, _):
        for k in range(H // 16):
            initb[j, pl.ds(k * 16, 16)] = row0[0, pl.ds(k * 16, 16)]
        return 0

    lax.fori_loop(0, EB, fill_body, 0)

    r0 = s * RPT
    for off, nr in ((0, 128), (128, 128), (256, 128), (384, 128), (512, 120)):
        pltpu.sync_copy(initb.at[pl.ds(0, nr)], h0_sp.at[pl.ds(r0 + off, nr)])
    plsc.subcore_barrier()

    dummy16 = jnp.full((16,), N, jnp.int32)
    for off, nr in ((0, 128), (128, 128), (256, 128), (384, 128), (512, 120)):
        v0 = r0 + off
        if nr < EB:
            idxb[pl.ds(112, 16)] = dummy16
        pltpu.sync_copy(sidx_hbm.at[pl.ds(v0, nr)], idxb.at[pl.ds(0, nr)])
        pltpu.sync_copy(emb_hbm.at[c, pl.ds(v0, nr)], rowb.at[pl.ds(0, nr)])
        pltpu.sync_copy(rowb, h0_sp.at[idxb])
    plsc.subcore_barrier()

    for off, nr in ((0, 128), (128, 128), (256, 128), (384, 128), (512, 120)):
        pltpu.sync_copy(h0_sp.at[pl.ds(r0 + off, nr)], rowb.at[pl.ds(0, nr)])
        pltpu.sync_copy(rowb.at[pl.ds(0, nr)],
                        h0_hbm.at[c, pl.ds(r0 + off, nr)])


@functools.partial(
    pl.kernel,
    mesh=_mesh,
    compiler_params=_sc_params,
    out_type=jax.ShapeDtypeStruct((2, NP, H), jnp.float32),
    scratch_types=[
        pltpu.VMEM((EB,), jnp.int32),
        pltpu.VMEM((EB,), jnp.int32),
        pltpu.VMEM((EB, H), jnp.float32),
        pltpu.VMEM((EB, H), jnp.float32),
        pltpu.VMEM_SHARED((NP, H), jnp.float32),
        pltpu.SemaphoreType.DMA,
    ],
)
def _sc_edge_pass(p_hbm, src_hbm, dst_hbm, acc_hbm,
                  sidxb, didxb, rowb, stageb, acc_sp, gsem):
    c = lax.axis_index("c")
    s = lax.axis_index("s")
    r0 = s * RPT
    for off, nr in ((0, 128), (128, 128), (256, 128), (384, 128), (512, 120)):
        pltpu.sync_copy(p_hbm.at[c, pl.ds(r0 + off, nr)],
                        stageb.at[pl.ds(0, nr)])
        pltpu.sync_copy(stageb.at[pl.ds(0, nr)],
                        acc_sp.at[pl.ds(r0 + off, nr)])
    plsc.subcore_barrier()

    ebase = s * EPT

    def edge_body(i, _):
        e0 = ebase + i * EB
        pltpu.sync_copy(src_hbm.at[pl.ds(e0, EB)], sidxb)
        pltpu.sync_copy(dst_hbm.at[pl.ds(e0, EB)], didxb)
        pltpu.async_copy(p_hbm.at[c].at[sidxb], rowb, gsem).wait()
        pltpu.sync_copy(rowb, acc_sp.at[didxb], add=True)
        return 0

    lax.fori_loop(0, NBATCH, edge_body, 0)
    plsc.subcore_barrier()

    for off, nr in ((0, 128), (128, 128), (256, 128), (384, 128), (512, 120)):
        pltpu.sync_copy(acc_sp.at[pl.ds(r0 + off, nr)],
                        stageb.at[pl.ds(0, nr)])
        pltpu.sync_copy(stageb.at[pl.ds(0, nr)],
                        acc_hbm.at[c, pl.ds(r0 + off, nr)])


def _tc_p1_body(h0_ref, w1_ref, dinv_ref, out_ref):
    acc = jnp.dot(h0_ref[0], w1_ref[0, 0], preferred_element_type=jnp.float32)
    acc = acc + jnp.dot(h0_ref[1], w1_ref[1, 0],
                        preferred_element_type=jnp.float32)
    out_ref[0] = dinv_ref[...] * acc


def _tc_p1(h0, w1s, dinvcol):
    return pl.pallas_call(
        _tc_p1_body,
        grid=(2, NB),
        in_specs=[
            pl.BlockSpec((2, 128, H), lambda c, r: (0, r, 0)),
            pl.BlockSpec((2, 1, H, H), lambda c, r: (0, c, 0, 0)),
            pl.BlockSpec((128, 1), lambda c, r: (r, 0)),
        ],
        out_specs=pl.BlockSpec((1, 128, H), lambda c, r: (c, r, 0)),
        out_shape=jax.ShapeDtypeStruct((2, NP, H), jnp.float32),
    )(h0, w1s, dinvcol)


def _tc_p2_body(acc_ref, w2_ref, dinv_ref, b1_ref, out_ref):
    dinv = dinv_ref[...]
    in0 = jnp.maximum(dinv * acc_ref[0] + b1_ref[0, pl.ds(0, H)], 0.0)
    in1 = jnp.maximum(dinv * acc_ref[1] + b1_ref[0, pl.ds(H, H)], 0.0)
    acc = jnp.dot(in0, w2_ref[0, 0], preferred_element_type=jnp.float32)
    acc = acc + jnp.dot(in1, w2_ref[1, 0], preferred_element_type=jnp.float32)
    out_ref[0] = dinv * acc


def _tc_p2(acc1, w2s, dinvcol, b1r):
    return pl.pallas_call(
        _tc_p2_body,
        grid=(2, NB),
        in_specs=[
            pl.BlockSpec((2, 128, H), lambda c, r: (0, r, 0)),
            pl.BlockSpec((2, 1, H, H), lambda c, r: (0, c, 0, 0)),
            pl.BlockSpec((128, 1), lambda c, r: (r, 0)),
            pl.BlockSpec((1, 128), lambda c, r: (0, 0)),
        ],
        out_specs=pl.BlockSpec((1, 128, H), lambda c, r: (c, r, 0)),
        out_shape=jax.ShapeDtypeStruct((2, NP, H), jnp.float32),
    )(acc1, w2s, dinvcol, b1r)


def _tc_final_body(acc_ref, dinv_ref, b2_ref, out_ref):
    dinv = dinv_ref[...]
    out_ref[:, 0:H] = dinv * acc_ref[0] + b2_ref[0, pl.ds(0, H)]
    out_ref[:, H:D] = dinv * acc_ref[1] + b2_ref[0, pl.ds(H, H)]


def _tc_final(acc2, dinvcol, b2r):
    return pl.pallas_call(
        _tc_final_body,
        grid=(NB,),
        in_specs=[
            pl.BlockSpec((2, 128, H), lambda r: (0, r, 0)),
            pl.BlockSpec((128, 1), lambda r: (r, 0)),
            pl.BlockSpec((1, 128), lambda r: (0, 0)),
        ],
        out_specs=pl.BlockSpec((128, D), lambda r: (r, 0)),
        out_shape=jax.ShapeDtypeStruct((NP, D), jnp.float32),
    )(acc2, dinvcol, b2r)


@jax.jit
def kernel(x, edge_index, emb, W1, b1, W2, b2):
    del x
    i32 = jnp.int32
    pad = jnp.full((E_PAD - E,), N, i32)
    src = jnp.concatenate([edge_index[0].astype(i32), pad])
    dst = jnp.concatenate([edge_index[1].astype(i32), pad])
    emb_pad = jnp.concatenate([emb, jnp.zeros((NP - N, D), jnp.float32)])
    emb_split = emb_pad.reshape(NP, 2, H).transpose(1, 0, 2)

    cs_part, cd_part = _sc_hist(src, dst)
    sidx2d, dinv2d = _tc_stats(
        cs_part.reshape(32, NB, 128), cd_part.reshape(32, NB, 128))
    sidx = sidx2d.reshape(NP)
    dinvcol = dinv2d.reshape(NP, 1)

    h0 = _sc_unique_gather(emb_split, sidx)

    w1s = W1.reshape(2, H, 2, H).transpose(0, 2, 1, 3)
    w2s = W2.reshape(2, H, 2, H).transpose(0, 2, 1, 3)
    b1r = b1.reshape(1, D)
    b2r = b2.reshape(1, D)

    p1 = _tc_p1(h0, w1s, dinvcol)
    acc1 = _sc_edge_pass(p1, src, dst)
    p2 = _tc_p2(acc1, w2s, dinvcol, b1r)
    acc2 = _sc_edge_pass(p2, src, dst)
    out = _tc_final(acc2, dinvcol, b2r)
    return out[:N]
